# baseline (speedup 1.0000x reference)
import functools
import jax
import jax.numpy as jnp
from jax import lax
from jax.experimental import pallas as pl
from jax.experimental.pallas import tpu as pltpu
from jax.experimental.pallas import tpu_sc as plsc

_ROWS = 2048
_COLS = 8192
_K = 64


import functools
import jax
import jax.numpy as jnp
from jax import lax
from jax.experimental import pallas as pl
from jax.experimental.pallas import tpu as pltpu
from jax.experimental.pallas import tpu_sc as plsc

_K = 64
_SC_ROWS = 512
_COLS = 8192
_NW = 32
_RPW = _SC_ROWS // _NW
_NV = _COLS // 16
_CAP = 512
_U = 8


def _scalar(v):
    return jnp.max(v) if getattr(v, "ndim", 0) else v


def _key(bb):
    return jnp.where(bb < 0, bb ^ jnp.int32(0x7FFFFFFF), bb)


def _sc_body(x_hbm, o_hbm, row_a, row_b, e_a, e_b, cand_v, bins_v,
             si_a, si_b, so_a, so_b):
    wid = lax.axis_index("s") * 2 + lax.axis_index("c")
    lanes = lax.iota(jnp.int32, 16)
    ones = jnp.full((16,), 1, jnp.int32)
    zero16 = jnp.full((16,), 0, jnp.int32)
    lane_cap = lanes * _CAP
    lane_bins = lanes * 64
    row0 = wid * _RPW

    def pick_bucket(k_rem):
        ts = []
        for j in range(4):
            acc = bins_v[pl.ds(16 * j, 16)]
            bins_v[pl.ds(16 * j, 16)] = zero16
            for l in range(1, 16):
                acc = acc + bins_v[pl.ds(l * 64 + 16 * j, 16)]
                bins_v[pl.ds(l * 64 + 16 * j, 16)] = zero16
            ts.append(acc)
        sums = [_scalar(jnp.sum(t)) for t in ts]
        rev = lambda t: lax.rev(t, (0,))
        b_dig = jnp.int32(-1)
        above = jnp.int32(0)
        suffix = jnp.int32(0)
        for j in (3, 2, 1, 0):
            ct = rev(plsc.cumsum(rev(ts[j]))) + suffix
            mask = ct >= k_rem
            cand_dig = jnp.where(mask, lanes + 16 * j, -1)
            bj = jnp.max(cand_dig)
            take = (b_dig < 0) & (bj >= 0)
            ct_b = jnp.max(jnp.where(cand_dig == bj, ct - ts[j], 0))
            b_dig = jnp.where(take, bj, b_dig)
            above = jnp.where(take, ct_b, above)
            suffix = suffix + sums[j]
        return b_dig, above

    def process_row(row_v, e_v):
        def hist_step(t, mvec):
            vs = [row_v[pl.ds(t * 16 * _U + q * 16, 16)] for q in range(_U)]
            bbs = [lax.bitcast_convert_type(v, jnp.int32) for v in vs]
            ds = [(_key(bb) >> 26) + 32 for bb in bbs]
            for q in range(_U):
                plsc.addupdate_scatter(bins_v, [lane_bins + ds[q]], ones)
            for q in range(_U):
                mvec = jnp.maximum(mvec, vs[q])
            return mvec

        mvec = lax.fori_loop(0, _NV // _U, hist_step,
                             jnp.full((16,), -jnp.inf, jnp.float32))
        m = jnp.max(mvec)

        b_dig, above = pick_bucket(jnp.int32(_K))
        k2 = jnp.int32(_K) - above

        base = (b_dig - 32) << 26
        base_v = jnp.broadcast_to(base, (16,))
        hi_v = base_v + jnp.int32(1 << 26)

        def comp_step(t, cnt_v):
            vs = [row_v[pl.ds(t * 16 * _U + q * 16, 16)] for q in range(_U)]
            ss = [_key(lax.bitcast_convert_type(v, jnp.int32)) for v in vs]
            mks = [(s >= base_v) & (s < hi_v) for s in ss]
            for q in range(_U):
                plsc.store_scatter(cand_v, [lane_cap + cnt_v], ss[q],
                                   mask=mks[q])
                cnt_v = cnt_v + jnp.where(mks[q], 1, 0)
            return cnt_v

        cnt_v = lax.fori_loop(0, _NV // _U, comp_step, zero16)
        nmax = _scalar(jnp.max(cnt_v))

        def hist2_step(j, c):
            ck = plsc.load_gather(cand_v, [lane_cap + j])
            d = (ck >> 20) & 63
            plsc.addupdate_scatter(bins_v, [lane_bins + d], ones,
                                   mask=j < cnt_v)
            return c

        lax.fori_loop(0, nmax, hist2_step, 0)
        b2, above2 = pick_bucket(k2)
        k3 = k2 - above2
        lo2_v = base_v | jnp.broadcast_to(b2 << 20, (16,))
        hi2_v = lo2_v + jnp.int32(1 << 20)

        def comp2_step(j, cnt2_v):
            ck = plsc.load_gather(cand_v, [lane_cap + j])
            mk = (ck >= lo2_v) & (ck < hi2_v) & (j < cnt_v)
            plsc.store_scatter(cand_v, [16 * _CAP + lane_cap + cnt2_v],
                               ck, mask=mk)
            return cnt2_v + jnp.where(mk, 1, 0)

        cnt2_v = lax.fori_loop(0, nmax, comp2_step, zero16)
        nmax2 = _scalar(jnp.max(cnt2_v))

        res = jnp.int32(0)
        lo2 = base | (b2 << 20)

        def count_ge(t_full_v):
            def cstep(j, acc_v):
                ck = plsc.load_gather(cand_v, [16 * _CAP + lane_cap + j])
                mm = (ck >= t_full_v) & (j < cnt2_v)
                return acc_v + jnp.where(mm, 1, 0)
            acc = lax.fori_loop(0, nmax2, cstep, zero16)
            return _scalar(jnp.sum(acc))

        for bit in range(19, -1, -1):
            cand_t = res | (1 << bit)
            cnt = count_ge(jnp.broadcast_to(lo2 | cand_t, (16,)))
            res = jnp.where(cnt >= k3, cand_t, res)

        t_key = jnp.broadcast_to(lo2 | res, (16,))
        t_bits = jnp.where(t_key < 0, t_key ^ jnp.int32(0x7FFFFFFF), t_key)
        thr = lax.bitcast_convert_type(t_bits, jnp.float32)

        mvec_b = jnp.broadcast_to(m, (16,))

        def exp_step(t, svec):
            vs = [row_v[pl.ds(t * 16 * _U + q * 16, 16)] for q in range(_U)]
            es = [jnp.where(v >= thr, jnp.exp(v - mvec_b), jnp.float32(0.0))
                  for v in vs]
            for q in range(_U):
                e_v[pl.ds(t * 16 * _U + q * 16, 16)] = es[q]
            for q in range(_U):
                svec = svec + es[q]
            return svec

        svec = lax.fori_loop(0, _NV // _U, exp_step,
                             jnp.full((16,), 0.0, jnp.float32))
        inv = jnp.float32(1.0) / jnp.broadcast_to(_scalar(jnp.sum(svec)), (16,))

        def scale_step(t, c):
            sls = [pl.ds(t * 16 * _U + q * 16, 16) for q in range(_U)]
            es = [e_v[sl] * inv for sl in sls]
            for q in range(_U):
                e_v[sls[q]] = es[q]
            return c

        lax.fori_loop(0, _NV // _U, scale_step, 0)

    for j in range(64):
        bins_v[pl.ds(16 * j, 16)] = zero16

    pltpu.async_copy(x_hbm.at[row0], row_a, si_a)
    pltpu.async_copy(x_hbm.at[row0 + 1], row_b, si_b)

    def do_pair(p, _):
        r0 = row0 + 2 * p

        pltpu.make_async_copy(x_hbm.at[r0], row_a, si_a).wait()

        @pl.when(p > 0)
        def _():
            pltpu.make_async_copy(e_a, o_hbm.at[r0], so_a).wait()

        process_row(row_a, e_a)

        @pl.when(2 * p + 2 < _RPW)
        def _():
            pltpu.async_copy(x_hbm.at[r0 + 2], row_a, si_a)

        pltpu.async_copy(e_a, o_hbm.at[r0], so_a)

        pltpu.make_async_copy(x_hbm.at[r0 + 1], row_b, si_b).wait()

        @pl.when(p > 0)
        def _():
            pltpu.make_async_copy(e_b, o_hbm.at[r0 + 1], so_b).wait()

        process_row(row_b, e_b)

        @pl.when(2 * p + 3 < _RPW)
        def _():
            pltpu.async_copy(x_hbm.at[r0 + 3], row_b, si_b)

        pltpu.async_copy(e_b, o_hbm.at[r0 + 1], so_b)
        return 0

    lax.fori_loop(0, _RPW // 2, do_pair, 0)
    pltpu.make_async_copy(e_a, o_hbm.at[row0], so_a).wait()
    pltpu.make_async_copy(e_b, o_hbm.at[row0 + 1], so_b).wait()


def _sc_call(x2):
    mesh = plsc.VectorSubcoreMesh(core_axis_name="c", subcore_axis_name="s")
    f = functools.partial(
        pl.kernel,
        mesh=mesh,
        compiler_params=pltpu.CompilerParams(needs_layout_passes=False),
        out_type=jax.ShapeDtypeStruct((_SC_ROWS, _COLS), jnp.float32),
        scratch_types=[
            pltpu.VMEM((_COLS,), jnp.float32),
            pltpu.VMEM((_COLS,), jnp.float32),
            pltpu.VMEM((_COLS,), jnp.float32),
            pltpu.VMEM((_COLS,), jnp.float32),
            pltpu.VMEM((2 * 16 * _CAP,), jnp.int32),
            pltpu.VMEM((1024,), jnp.int32),
            pltpu.SemaphoreType.DMA,
            pltpu.SemaphoreType.DMA,
            pltpu.SemaphoreType.DMA,
            pltpu.SemaphoreType.DMA,
        ],
    )(_sc_body)
    return f(x2)


_TC_ROWS = _ROWS - _SC_ROWS
_BLOCK_R = 256


def _pack_i16(x32):
    return pltpu.bitcast(x32, jnp.int16)


def _pack_i32(x16):
    return pltpu.bitcast(x16, jnp.int32)


def _count_pair(cmp):
    c16 = cmp.astype(jnp.int16)
    c32 = _pack_i32(c16)
    return jnp.sum(c32, axis=-1, keepdims=True)


def _lo_bit(bit):
    return jnp.int32(1 << bit)


def _hi_bit(bit):
    return jnp.int32((1 << (bit + 16)) - (1 << 32 if bit == 15 else 0))


_BIAS = -2147450880


def _tc_body(x_ref, o_ref):
    x = x_ref[...]
    r2 = x.shape[0] // 2
    b = jax.lax.bitcast_convert_type(x, jnp.int32)
    u = jax.lax.bitcast_convert_type(x, jnp.uint32)
    key = jnp.where(b < 0, ~u, u | jnp.uint32(0x80000000))
    m = jnp.max(x, axis=-1, keepdims=True)

    hib = ((key >> 16) ^ jnp.uint32(0x8000)).astype(jnp.int16)
    res1 = jnp.zeros((r2, 1), jnp.int32)
    for bit in range(15, -1, -1):
        cand = res1 | (_lo_bit(bit) | _hi_bit(bit))
        cand_b = _pack_i16(cand ^ _BIAS)
        cmp = hib >= cand_b
        s = _count_pair(cmp)
        ge_lo = (s & 0xFFFF) >= _K
        ge_hi = jax.lax.shift_right_logical(s, 16) >= _K
        res1 = (res1
                | jnp.where(ge_lo, _lo_bit(bit), 0)
                | jnp.where(ge_hi, _hi_bit(bit), 0))
    res1_b = _pack_i16(res1 ^ _BIAS)
    s = _count_pair(hib > res1_b)
    k2_lo = _K - (s & 0xFFFF)
    k2_hi = _K - jax.lax.shift_right_logical(s, 16)

    boundary = hib == res1_b
    lob = jnp.where(
        boundary,
        ((key ^ jnp.uint32(0x8000)) & jnp.uint32(0xFFFF)).astype(jnp.int16),
        jnp.int16(-32768))
    res2 = jnp.zeros((r2, 1), jnp.int32)
    for bit in range(15, -1, -1):
        cand = res2 | (_lo_bit(bit) | _hi_bit(bit))
        cand_b = _pack_i16(cand ^ _BIAS)
        cmp = lob >= cand_b
        s = _count_pair(cmp)
        ge_lo = (s & 0xFFFF) >= k2_lo
        ge_hi = jax.lax.shift_right_logical(s, 16) >= k2_hi
        res2 = (res2
                | jnp.where(ge_lo, _lo_bit(bit), 0)
                | jnp.where(ge_hi, _hi_bit(bit), 0))

    hi16 = _pack_i16(res1).astype(jnp.int32) & 0xFFFF
    lo16 = _pack_i16(res2).astype(jnp.int32) & 0xFFFF
    T = jax.lax.bitcast_convert_type((hi16 << 16) | lo16, jnp.uint32)
    was_nonneg = (T & jnp.uint32(0x80000000)) != 0
    ub = jnp.where(was_nonneg, T & jnp.uint32(0x7FFFFFFF), ~T)
    thresh = jax.lax.bitcast_convert_type(ub, jnp.float32)

    e = jnp.where(x >= thresh, jnp.exp(x - m), jnp.float32(0.0))
    s = jnp.sum(e, axis=-1, keepdims=True)
    o_ref[...] = e / s


def _tc_call(x2):
    n = x2.shape[0]
    return pl.pallas_call(
        _tc_body,
        grid=(n // _BLOCK_R,),
        in_specs=[pl.BlockSpec((_BLOCK_R, _COLS), lambda i: (i, 0))],
        out_specs=pl.BlockSpec((_BLOCK_R, _COLS), lambda i: (i, 0)),
        out_shape=jax.ShapeDtypeStruct((n, _COLS), jnp.float32),
    )(x2)


def kernel(X):
    shape = X.shape
    x2 = X.reshape(_ROWS, _COLS)
    sc_out = _sc_call(x2[_TC_ROWS:])
    half = _TC_ROWS // 2
    tc_out0 = _tc_call(x2[:half])
    tc_out1 = _tc_call(x2[half:_TC_ROWS])
    out = jnp.concatenate([tc_out0, tc_out1, sc_out], axis=0)
    return out.reshape(shape)


# speedup vs baseline: 1.1077x; 1.1077x over previous
import functools
import jax
import jax.numpy as jnp
from jax import lax
from jax.experimental import pallas as pl
from jax.experimental.pallas import tpu as pltpu
from jax.experimental.pallas import tpu_sc as plsc

_ROWS = 2048
_COLS = 8192
_K = 64


import functools
import jax
import jax.numpy as jnp
from jax import lax
from jax.experimental import pallas as pl
from jax.experimental.pallas import tpu as pltpu
from jax.experimental.pallas import tpu_sc as plsc

_K = 64
_SC_ROWS = 512
_COLS = 8192
_NW = 32
_RPW = _SC_ROWS // _NW
_NV = _COLS // 16
_CAP = 512
_U = 8


def _scalar(v):
    return jnp.max(v) if getattr(v, "ndim", 0) else v


def _key(bb):
    return jnp.where(bb < 0, bb ^ jnp.int32(0x7FFFFFFF), bb)


def _sc_body(x_hbm, o_hbm, row_a, row_b, e_a, e_b, cand_v, bins_v,
             si_a, si_b, so_a, so_b):
    wid = lax.axis_index("s") * 2 + lax.axis_index("c")
    lanes = lax.iota(jnp.int32, 16)
    ones = jnp.full((16,), 1, jnp.int32)
    zero16 = jnp.full((16,), 0, jnp.int32)
    lane_cap = lanes * _CAP
    lane_bins = lanes * 64
    row0 = wid * _RPW

    def pick_bucket(k_rem):
        ts = []
        for j in range(4):
            acc = bins_v[pl.ds(16 * j, 16)]
            bins_v[pl.ds(16 * j, 16)] = zero16
            for l in range(1, 16):
                acc = acc + bins_v[pl.ds(l * 64 + 16 * j, 16)]
                bins_v[pl.ds(l * 64 + 16 * j, 16)] = zero16
            ts.append(acc)
        sums = [_scalar(jnp.sum(t)) for t in ts]
        rev = lambda t: lax.rev(t, (0,))
        b_dig = jnp.int32(-1)
        above = jnp.int32(0)
        suffix = jnp.int32(0)
        for j in (3, 2, 1, 0):
            ct = rev(plsc.cumsum(rev(ts[j]))) + suffix
            mask = ct >= k_rem
            cand_dig = jnp.where(mask, lanes + 16 * j, -1)
            bj = jnp.max(cand_dig)
            take = (b_dig < 0) & (bj >= 0)
            ct_b = jnp.max(jnp.where(cand_dig == bj, ct - ts[j], 0))
            b_dig = jnp.where(take, bj, b_dig)
            above = jnp.where(take, ct_b, above)
            suffix = suffix + sums[j]
        return b_dig, above

    def process_row(row_v, e_v):
        def hist_step(t, mvec):
            vs = [row_v[pl.ds(t * 16 * _U + q * 16, 16)] for q in range(_U)]
            bbs = [lax.bitcast_convert_type(v, jnp.int32) for v in vs]
            ds = [(_key(bb) >> 26) + 32 for bb in bbs]
            for q in range(_U):
                plsc.addupdate_scatter(bins_v, [lane_bins + ds[q]], ones)
            for q in range(_U):
                mvec = jnp.maximum(mvec, vs[q])
            return mvec

        mvec = lax.fori_loop(0, _NV // _U, hist_step,
                             jnp.full((16,), -jnp.inf, jnp.float32))
        m = jnp.max(mvec)

        b_dig, above = pick_bucket(jnp.int32(_K))
        k2 = jnp.int32(_K) - above

        base = (b_dig - 32) << 26
        base_v = jnp.broadcast_to(base, (16,))
        hi_v = base_v + jnp.int32(1 << 26)

        def comp_step(t, cnt_v):
            vs = [row_v[pl.ds(t * 16 * _U + q * 16, 16)] for q in range(_U)]
            ss = [_key(lax.bitcast_convert_type(v, jnp.int32)) for v in vs]
            mks = [(s >= base_v) & (s < hi_v) for s in ss]
            for q in range(_U):
                plsc.store_scatter(cand_v, [lane_cap + cnt_v], ss[q],
                                   mask=mks[q])
                cnt_v = cnt_v + jnp.where(mks[q], 1, 0)
            return cnt_v

        cnt_v = lax.fori_loop(0, _NV // _U, comp_step, zero16)
        nmax = _scalar(jnp.max(cnt_v))

        def hist2_step(j, c):
            ck = plsc.load_gather(cand_v, [lane_cap + j])
            d = (ck >> 20) & 63
            plsc.addupdate_scatter(bins_v, [lane_bins + d], ones,
                                   mask=j < cnt_v)
            return c

        lax.fori_loop(0, nmax, hist2_step, 0)
        b2, above2 = pick_bucket(k2)
        k3 = k2 - above2
        lo2_v = base_v | jnp.broadcast_to(b2 << 20, (16,))
        hi2_v = lo2_v + jnp.int32(1 << 20)

        def comp2_step(j, cnt2_v):
            ck = plsc.load_gather(cand_v, [lane_cap + j])
            mk = (ck >= lo2_v) & (ck < hi2_v) & (j < cnt_v)
            plsc.store_scatter(cand_v, [16 * _CAP + lane_cap + cnt2_v],
                               ck, mask=mk)
            return cnt2_v + jnp.where(mk, 1, 0)

        cnt2_v = lax.fori_loop(0, nmax, comp2_step, zero16)
        nmax2 = _scalar(jnp.max(cnt2_v))

        res = jnp.int32(0)
        lo2 = base | (b2 << 20)

        def count_ge(t_full_v):
            def cstep(j, acc_v):
                ck = plsc.load_gather(cand_v, [16 * _CAP + lane_cap + j])
                mm = (ck >= t_full_v) & (j < cnt2_v)
                return acc_v + jnp.where(mm, 1, 0)
            acc = lax.fori_loop(0, nmax2, cstep, zero16)
            return _scalar(jnp.sum(acc))

        for bit in range(19, -1, -1):
            cand_t = res | (1 << bit)
            cnt = count_ge(jnp.broadcast_to(lo2 | cand_t, (16,)))
            res = jnp.where(cnt >= k3, cand_t, res)

        t_key = jnp.broadcast_to(lo2 | res, (16,))
        t_bits = jnp.where(t_key < 0, t_key ^ jnp.int32(0x7FFFFFFF), t_key)
        thr = lax.bitcast_convert_type(t_bits, jnp.float32)

        mvec_b = jnp.broadcast_to(m, (16,))

        def exp_step(t, svec):
            vs = [row_v[pl.ds(t * 16 * _U + q * 16, 16)] for q in range(_U)]
            es = [jnp.where(v >= thr, jnp.exp(v - mvec_b), jnp.float32(0.0))
                  for v in vs]
            for q in range(_U):
                e_v[pl.ds(t * 16 * _U + q * 16, 16)] = es[q]
            for q in range(_U):
                svec = svec + es[q]
            return svec

        svec = lax.fori_loop(0, _NV // _U, exp_step,
                             jnp.full((16,), 0.0, jnp.float32))
        inv = jnp.float32(1.0) / jnp.broadcast_to(_scalar(jnp.sum(svec)), (16,))

        def scale_step(t, c):
            sls = [pl.ds(t * 16 * _U + q * 16, 16) for q in range(_U)]
            es = [e_v[sl] * inv for sl in sls]
            for q in range(_U):
                e_v[sls[q]] = es[q]
            return c

        lax.fori_loop(0, _NV // _U, scale_step, 0)

    for j in range(64):
        bins_v[pl.ds(16 * j, 16)] = zero16

    pltpu.async_copy(x_hbm.at[row0], row_a, si_a)
    pltpu.async_copy(x_hbm.at[row0 + 1], row_b, si_b)

    def do_pair(p, _):
        r0 = row0 + 2 * p

        pltpu.make_async_copy(x_hbm.at[r0], row_a, si_a).wait()

        @pl.when(p > 0)
        def _():
            pltpu.make_async_copy(e_a, o_hbm.at[r0], so_a).wait()

        process_row(row_a, e_a)

        @pl.when(2 * p + 2 < _RPW)
        def _():
            pltpu.async_copy(x_hbm.at[r0 + 2], row_a, si_a)

        pltpu.async_copy(e_a, o_hbm.at[r0], so_a)

        pltpu.make_async_copy(x_hbm.at[r0 + 1], row_b, si_b).wait()

        @pl.when(p > 0)
        def _():
            pltpu.make_async_copy(e_b, o_hbm.at[r0 + 1], so_b).wait()

        process_row(row_b, e_b)

        @pl.when(2 * p + 3 < _RPW)
        def _():
            pltpu.async_copy(x_hbm.at[r0 + 3], row_b, si_b)

        pltpu.async_copy(e_b, o_hbm.at[r0 + 1], so_b)
        return 0

    lax.fori_loop(0, _RPW // 2, do_pair, 0)
    pltpu.make_async_copy(e_a, o_hbm.at[row0], so_a).wait()
    pltpu.make_async_copy(e_b, o_hbm.at[row0 + 1], so_b).wait()


def _sc_call(x2):
    mesh = plsc.VectorSubcoreMesh(core_axis_name="c", subcore_axis_name="s")
    f = functools.partial(
        pl.kernel,
        mesh=mesh,
        compiler_params=pltpu.CompilerParams(needs_layout_passes=False),
        out_type=jax.ShapeDtypeStruct((_SC_ROWS, _COLS), jnp.float32),
        scratch_types=[
            pltpu.VMEM((_COLS,), jnp.float32),
            pltpu.VMEM((_COLS,), jnp.float32),
            pltpu.VMEM((_COLS,), jnp.float32),
            pltpu.VMEM((_COLS,), jnp.float32),
            pltpu.VMEM((2 * 16 * _CAP,), jnp.int32),
            pltpu.VMEM((1024,), jnp.int32),
            pltpu.SemaphoreType.DMA,
            pltpu.SemaphoreType.DMA,
            pltpu.SemaphoreType.DMA,
            pltpu.SemaphoreType.DMA,
        ],
    )(_sc_body)
    return f(x2)


_TC_ROWS = _ROWS - _SC_ROWS
_BLOCK_R = 256


def _pack_i16(x32):
    return pltpu.bitcast(x32, jnp.int16)


def _pack_i32(x16):
    return pltpu.bitcast(x16, jnp.int32)


def _count_pair(cmp):
    c16 = cmp.astype(jnp.int16)
    c32 = _pack_i32(c16)
    return jnp.sum(c32, axis=-1, keepdims=True)


def _lo_bit(bit):
    return jnp.int32(1 << bit)


def _hi_bit(bit):
    return jnp.int32((1 << (bit + 16)) - (1 << 32 if bit == 15 else 0))


_BIAS = -2147450880


def _tc_body(x_ref, o_ref):
    x = x_ref[...]
    r2 = x.shape[0] // 2
    b = jax.lax.bitcast_convert_type(x, jnp.int32)
    u = jax.lax.bitcast_convert_type(x, jnp.uint32)
    key = jnp.where(b < 0, ~u, u | jnp.uint32(0x80000000))
    m = jnp.max(x, axis=-1, keepdims=True)

    hib = ((key >> 16) ^ jnp.uint32(0x8000)).astype(jnp.int16)
    res1 = jnp.zeros((r2, 1), jnp.int32)
    for bit in range(15, -1, -1):
        cand = res1 | (_lo_bit(bit) | _hi_bit(bit))
        cand_b = _pack_i16(cand ^ _BIAS)
        cmp = hib >= cand_b
        s = _count_pair(cmp)
        ge_lo = (s & 0xFFFF) >= _K
        ge_hi = jax.lax.shift_right_logical(s, 16) >= _K
        res1 = (res1
                | jnp.where(ge_lo, _lo_bit(bit), 0)
                | jnp.where(ge_hi, _hi_bit(bit), 0))
    res1_b = _pack_i16(res1 ^ _BIAS)
    s = _count_pair(hib > res1_b)
    k2_lo = _K - (s & 0xFFFF)
    k2_hi = _K - jax.lax.shift_right_logical(s, 16)

    boundary = hib == res1_b
    lob = jnp.where(
        boundary,
        ((key ^ jnp.uint32(0x8000)) & jnp.uint32(0xFFFF)).astype(jnp.int16),
        jnp.int16(-32768))
    res2 = jnp.zeros((r2, 1), jnp.int32)
    for bit in range(15, -1, -1):
        cand = res2 | (_lo_bit(bit) | _hi_bit(bit))
        cand_b = _pack_i16(cand ^ _BIAS)
        cmp = lob >= cand_b
        s = _count_pair(cmp)
        ge_lo = (s & 0xFFFF) >= k2_lo
        ge_hi = jax.lax.shift_right_logical(s, 16) >= k2_hi
        res2 = (res2
                | jnp.where(ge_lo, _lo_bit(bit), 0)
                | jnp.where(ge_hi, _hi_bit(bit), 0))

    hi16 = _pack_i16(res1).astype(jnp.int32) & 0xFFFF
    lo16 = _pack_i16(res2).astype(jnp.int32) & 0xFFFF
    T = jax.lax.bitcast_convert_type((hi16 << 16) | lo16, jnp.uint32)
    was_nonneg = (T & jnp.uint32(0x80000000)) != 0
    ub = jnp.where(was_nonneg, T & jnp.uint32(0x7FFFFFFF), ~T)
    thresh = jax.lax.bitcast_convert_type(ub, jnp.float32)

    e = jnp.where(x >= thresh, jnp.exp(x - m), jnp.float32(0.0))
    s = jnp.sum(e, axis=-1, keepdims=True)
    o_ref[...] = e / s


def _tc_call(x2):
    n = x2.shape[0]
    return pl.pallas_call(
        _tc_body,
        grid=(n // _BLOCK_R,),
        in_specs=[pl.BlockSpec((_BLOCK_R, _COLS), lambda i: (i, 0))],
        out_specs=pl.BlockSpec((_BLOCK_R, _COLS), lambda i: (i, 0)),
        out_shape=jax.ShapeDtypeStruct((n, _COLS), jnp.float32),
    )(x2)


def kernel(X):
    shape = X.shape
    x2 = X.reshape(_ROWS, _COLS)
    sc_out = _sc_call(x2[_TC_ROWS:])
    tc_out = _tc_call(x2[:_TC_ROWS])
    out = jnp.concatenate([tc_out, sc_out], axis=0)
    return out.reshape(shape)
